# baseline (speedup 1.0000x reference)
import functools

import jax
import jax.numpy as jnp
from jax.experimental import pallas as pl
from jax.experimental.pallas import tpu as pltpu
from jax.experimental.pallas import tpu_sc as plsc

B, T, D = 4, 8192, 2048


def _make_sc_gather():
    mesh = plsc.ScalarSubcoreMesh(axis_name="c", num_cores=1)

    @functools.partial(
        pl.kernel,
        mesh=mesh,
        out_type=jax.ShapeDtypeStruct((B * D,), jnp.float32),
        scratch_types=[
            pltpu.SMEM((B,), jnp.int32),
            pltpu.SemaphoreType.DMA,
        ],
        compiler_params=pltpu.CompilerParams(needs_layout_passes=False),
    )
    def sc_gather(feat_hbm, len_hbm, out_hbm, len_s, sem):
        pltpu.sync_copy(len_hbm, len_s)
        for b in range(B):
            row = b * T + len_s[b] - 1
            pltpu.async_copy(feat_hbm.at[row], out_hbm.at[pl.ds(b * D, D)], sem)
        pltpu.make_async_copy(out_hbm, out_hbm, sem).wait()

    return sc_gather


_sc_gather = _make_sc_gather()


def kernel(features, _mask, lengths):
    feat = features.reshape(B * T, D)
    out = _sc_gather(feat, lengths)
    return out.reshape(B, 1, D)
